# baseline (speedup 1.0000x reference)
import functools

import jax
import jax.numpy as jnp
from jax import lax
from jax.experimental import pallas as pl
from jax.experimental.pallas import tpu as pltpu
from jax.experimental.pallas import tpu_sc as plsc

N_NODES = 10000
N_EDGES = 320000
D = 128

NC = 2
NS = 16
NW = NC * NS

GROUP = 128
G = 80
E_PAD = NW * G * GROUP
N_ACC = 10240
N_DUMMY = N_ACC - N_NODES
PHASES = 2
GP = G // PHASES

ROWS_INIT = N_ACC // NS
ROWS_OUT = 624


def _sc_body(x_hbm, src_hbm, dst_hbm, part_hbm, src_v, dst_v, rows_v,
             acc_sh, gsem, ssem):
    c = lax.axis_index("c")
    s = lax.axis_index("s")
    wid = c * NS + s

    zeros16 = jnp.zeros((16,), jnp.float32)

    def zrow(r, _):
        def zcol(k, _):
            rows_v[0, r, pl.ds(k * 16, 16)] = zeros16
            return 0
        return lax.fori_loop(0, D // 16, zcol, 0)

    lax.fori_loop(0, GROUP, zrow, 0)

    base = s * ROWS_INIT
    for j in range(ROWS_INIT // GROUP):
        pltpu.sync_copy(rows_v.at[0], acc_sh.at[pl.ds(base + j * GROUP, GROUP)])

    plsc.subcore_barrier()

    H = 4
    SUB = GROUP // H

    def gather_group(j, buf):
        for h in range(H):
            pltpu.async_copy(x_hbm.at[src_v.at[j, pl.ds(h * SUB, SUB)]],
                             rows_v.at[buf, pl.ds(h * SUB, SUB)], gsem)

    def gather_wait():
        for _ in range(H):
            pltpu.make_async_copy(x_hbm.at[pl.ds(0, SUB)],
                                  rows_v.at[0, pl.ds(0, SUB)], gsem).wait()

    def scatter_wait():
        pltpu.make_async_copy(x_hbm.at[pl.ds(0, GROUP)], rows_v.at[0],
                              ssem).wait()

    for phase in range(PHASES):
        pltpu.sync_copy(src_hbm.at[wid, pl.ds(phase * GP, GP)], src_v)
        pltpu.sync_copy(dst_hbm.at[wid, pl.ds(phase * GP, GP)], dst_v)

        gather_group(0, 0)

        def edge_body(j, _):
            cur = lax.rem(j, 2)
            nxt = lax.rem(j + 1, 2)
            gather_wait()

            @pl.when(j >= 1)
            def _():
                scatter_wait()

            @pl.when(j + 1 < GP)
            def _():
                gather_group(j + 1, nxt)

            pltpu.async_copy(rows_v.at[cur], acc_sh.at[dst_v.at[j]], ssem,
                             add=True)
            return 0

        lax.fori_loop(0, GP, edge_body, 0)
        scatter_wait()

    plsc.subcore_barrier()

    ob = s * ROWS_OUT
    cnt = N_NODES - (NS - 1) * ROWS_OUT

    @pl.when(s < NS - 1)
    def _():
        pltpu.sync_copy(acc_sh.at[pl.ds(ob, ROWS_OUT)],
                        part_hbm.at[c, pl.ds(ob, ROWS_OUT)])

    @pl.when(s == NS - 1)
    def _():
        pltpu.sync_copy(acc_sh.at[pl.ds((NS - 1) * ROWS_OUT, cnt)],
                        part_hbm.at[c, pl.ds((NS - 1) * ROWS_OUT, cnt)])


@jax.jit
def _sc_scatter(x, src_r, dst_r):
    mesh = plsc.VectorSubcoreMesh(core_axis_name="c", subcore_axis_name="s",
                                  num_cores=NC, num_subcores=NS)
    return pl.kernel(
        _sc_body,
        out_type=jax.ShapeDtypeStruct((NC, N_NODES, D), jnp.float32),
        mesh=mesh,
        scratch_types=[
            pltpu.VMEM((GP, GROUP), jnp.int32),
            pltpu.VMEM((GP, GROUP), jnp.int32),
            pltpu.VMEM((2, GROUP, D), jnp.float32),
            pltpu.VMEM_SHARED((N_ACC, D), jnp.float32),
            pltpu.SemaphoreType.DMA,
            pltpu.SemaphoreType.DMA,
        ],
    )(x, src_r, dst_r)


def _combine_body(scale_ref, x_ref, p_ref, o_ref):
    o_ref[...] = scale_ref[0, 0] * x_ref[...] + p_ref[0] + p_ref[1]


@jax.jit
def _combine(scale, x, partials):
    blk = 1000
    grid = N_NODES // blk
    return pl.pallas_call(
        _combine_body,
        out_shape=jax.ShapeDtypeStruct((N_NODES, D), jnp.float32),
        grid=(grid,),
        in_specs=[
            pl.BlockSpec((1, 1), lambda i: (0, 0)),
            pl.BlockSpec((blk, D), lambda i: (i, 0)),
            pl.BlockSpec((NC, blk, D), lambda i: (0, i, 0)),
        ],
        out_specs=pl.BlockSpec((blk, D), lambda i: (i, 0)),
    )(scale, x, partials)


def kernel(x, edge_index, eps):
    src = edge_index[0]
    dst = edge_index[1]
    pad = E_PAD - N_EDGES
    pad_src = (jnp.arange(pad, dtype=jnp.int32) * 89) % N_NODES
    pad_dst = N_NODES + (jnp.arange(pad, dtype=jnp.int32) % N_DUMMY)
    src_r = jnp.concatenate([src, pad_src]).reshape(NW, G, GROUP)
    dst_r = jnp.concatenate([dst, pad_dst]).reshape(NW, G, GROUP)
    partials = _sc_scatter(x, src_r, dst_r)
    scale = (1.0 + eps).reshape(1, 1)
    return _combine(scale, x, partials)


# speedup vs baseline: 1.0132x; 1.0132x over previous
import functools

import jax
import jax.numpy as jnp
from jax import lax
from jax.experimental import pallas as pl
from jax.experimental.pallas import tpu as pltpu
from jax.experimental.pallas import tpu_sc as plsc

N_NODES = 10000
N_EDGES = 320000
D = 128

NC = 2
NS = 16
NW = NC * NS

GROUP = 128
G = 80
E_PAD = NW * G * GROUP
N_ACC = 10240
N_DUMMY = N_ACC - N_NODES
PHASES = 2
GP = G // PHASES

H = 4
SUB = GROUP // H

ROWS_INIT = N_ACC // NS
ROWS_OUT = 624


def _sc_body(x_hbm, src_hbm, dst_hbm, part_hbm, src_v, dst_v, rows_v,
             acc_sh, gsem, ssem, isem):
    c = lax.axis_index("c")
    s = lax.axis_index("s")
    wid = c * NS + s

    pltpu.async_copy(src_hbm.at[wid, pl.ds(0, GP)], src_v, isem)
    pltpu.async_copy(dst_hbm.at[wid, pl.ds(0, GP)], dst_v, isem)

    zeros16 = jnp.zeros((16,), jnp.float32)

    def zrow(r, _):
        def zcol(k, _):
            rows_v[0, r, pl.ds(k * 16, 16)] = zeros16
            return 0
        return lax.fori_loop(0, D // 16, zcol, 0)

    lax.fori_loop(0, GROUP, zrow, 0)

    base = s * ROWS_INIT
    for j in range(ROWS_INIT // GROUP):
        pltpu.async_copy(rows_v.at[0],
                         acc_sh.at[pl.ds(base + j * GROUP, GROUP)], isem)
    for j in range(ROWS_INIT // GROUP):
        pltpu.make_async_copy(rows_v.at[0],
                              acc_sh.at[pl.ds(base, GROUP)], isem).wait()

    pltpu.make_async_copy(src_hbm.at[0, pl.ds(0, GP)], src_v, isem).wait()
    pltpu.make_async_copy(src_hbm.at[0, pl.ds(0, GP)], dst_v, isem).wait()

    plsc.subcore_barrier()

    def gather_group(j, buf):
        for h in range(H):
            pltpu.async_copy(x_hbm.at[src_v.at[j, pl.ds(h * SUB, SUB)]],
                             rows_v.at[buf, pl.ds(h * SUB, SUB)], gsem)

    def gather_wait():
        for _ in range(H):
            pltpu.make_async_copy(x_hbm.at[pl.ds(0, SUB)],
                                  rows_v.at[0, pl.ds(0, SUB)], gsem).wait()

    def scatter_wait():
        pltpu.make_async_copy(x_hbm.at[pl.ds(0, GROUP)], rows_v.at[0],
                              ssem).wait()

    def run_phase():
        gather_group(0, 0)

        def edge_body(j, _):
            cur = lax.rem(j, 2)
            nxt = lax.rem(j + 1, 2)
            gather_wait()

            @pl.when(j >= 1)
            def _():
                scatter_wait()

            @pl.when(j + 1 < GP)
            def _():
                gather_group(j + 1, nxt)

            pltpu.async_copy(rows_v.at[cur], acc_sh.at[dst_v.at[j]], ssem,
                             add=True)
            return 0

        lax.fori_loop(0, GP, edge_body, 0)
        scatter_wait()

    run_phase()

    pltpu.sync_copy(src_hbm.at[wid, pl.ds(GP, GP)], src_v)
    pltpu.sync_copy(dst_hbm.at[wid, pl.ds(GP, GP)], dst_v)
    run_phase()

    plsc.subcore_barrier()

    ob = s * ROWS_OUT
    cnt = N_NODES - (NS - 1) * ROWS_OUT

    @pl.when(s < NS - 1)
    def _():
        pltpu.sync_copy(acc_sh.at[pl.ds(ob, ROWS_OUT)],
                        part_hbm.at[c, pl.ds(ob, ROWS_OUT)])

    @pl.when(s == NS - 1)
    def _():
        pltpu.sync_copy(acc_sh.at[pl.ds((NS - 1) * ROWS_OUT, cnt)],
                        part_hbm.at[c, pl.ds((NS - 1) * ROWS_OUT, cnt)])


@jax.jit
def _sc_scatter(x, src_r, dst_r):
    mesh = plsc.VectorSubcoreMesh(core_axis_name="c", subcore_axis_name="s",
                                  num_cores=NC, num_subcores=NS)
    return pl.kernel(
        _sc_body,
        out_type=jax.ShapeDtypeStruct((NC, N_NODES, D), jnp.float32),
        mesh=mesh,
        scratch_types=[
            pltpu.VMEM((GP, GROUP), jnp.int32),
            pltpu.VMEM((GP, GROUP), jnp.int32),
            pltpu.VMEM((2, GROUP, D), jnp.float32),
            pltpu.VMEM_SHARED((N_ACC, D), jnp.float32),
            pltpu.SemaphoreType.DMA,
            pltpu.SemaphoreType.DMA,
            pltpu.SemaphoreType.DMA,
        ],
    )(x, src_r, dst_r)


def _combine_body(scale_ref, x_ref, p_ref, o_ref):
    o_ref[...] = scale_ref[0, 0] * x_ref[...] + p_ref[0] + p_ref[1]


@jax.jit
def _combine(scale, x, partials):
    blk = 1000
    grid = N_NODES // blk
    return pl.pallas_call(
        _combine_body,
        out_shape=jax.ShapeDtypeStruct((N_NODES, D), jnp.float32),
        grid=(grid,),
        in_specs=[
            pl.BlockSpec((1, 1), lambda i: (0, 0)),
            pl.BlockSpec((blk, D), lambda i: (i, 0)),
            pl.BlockSpec((NC, blk, D), lambda i: (0, i, 0)),
        ],
        out_specs=pl.BlockSpec((blk, D), lambda i: (i, 0)),
    )(scale, x, partials)


def kernel(x, edge_index, eps):
    src = edge_index[0]
    dst = edge_index[1]
    pad = E_PAD - N_EDGES
    pad_src = (jnp.arange(pad, dtype=jnp.int32) * 89) % N_NODES
    pad_dst = N_NODES + (jnp.arange(pad, dtype=jnp.int32) % N_DUMMY)
    src_r = jnp.concatenate([src, pad_src]).reshape(NW, G, GROUP)
    dst_r = jnp.concatenate([dst, pad_dst]).reshape(NW, G, GROUP)
    partials = _sc_scatter(x, src_r, dst_r)
    scale = (1.0 + eps).reshape(1, 1)
    return _combine(scale, x, partials)
